# baseline (speedup 1.0000x reference)
import functools

import jax
import jax.numpy as jnp
from jax import lax
from jax.experimental import pallas as pl
from jax.experimental.pallas import tpu as pltpu
from jax.experimental.pallas import tpu_sc as plsc

_INTERVAL = [200000, 150000, 150000, 100000, 100000, 100000, 100000, 50000, 50000]
_OFFS = tuple(sum(_INTERVAL[:j]) for j in range(len(_INTERVAL)))

_B = 16384
_D = 32
_K = 9
_NC = 2
_NS = 16
_NW = _NC * _NS
_BPW = _B // _NW
_CHUNK = 128
_NCH = _BPW // _CHUNK
_L = 16


def _body(data_hbm, table_hbm, attn_hbm, out_hbm,
          data_v, idx_v, rows_v, out_v, attn_v, sem):
    wid = lax.axis_index("s") * _NC + lax.axis_index("c")
    base = wid * _BPW

    pltpu.sync_copy(data_hbm.at[pl.ds(base * _K, _BPW * _K)], data_v)
    pltpu.sync_copy(attn_hbm, attn_v)

    iota = lax.iota(jnp.int32, _L)
    av = attn_v[...]
    w = [jnp.full((_L,), jnp.sum(jnp.where(iota == j, av, 0.0)), jnp.float32)
         for j in range(_K)]

    def chunk_body(g, carry):
        cb = g * _CHUNK
        for j in range(_K):
            for q in range(_CHUNK // _L):
                fvec = (cb + 16 * q + iota) * _K + j
                dv = plsc.load_gather(data_v, [fvec])
                idx_v[j, pl.ds(16 * q, _L)] = dv + _OFFS[j]
        cps = [pltpu.async_copy(table_hbm.at[idx_v.at[j]], rows_v.at[j], sem)
               for j in range(_K)]
        for c in cps:
            c.wait()
        def b_body(b, c2):
            for h in range(_D // _L):
                acc = rows_v[0, b, pl.ds(16 * h, _L)] * w[0]
                for j in range(1, _K):
                    acc = acc + rows_v[j, b, pl.ds(16 * h, _L)] * w[j]
                out_v[cb + b, pl.ds(16 * h, _L)] = acc
            return c2
        lax.fori_loop(0, _CHUNK, b_body, 0)
        return carry

    lax.fori_loop(0, _NCH, chunk_body, 0)
    pltpu.sync_copy(out_v, out_hbm.at[pl.ds(base, _BPW)])


@jax.jit
def _emb(data_flat, table, attn16):
    mesh = plsc.VectorSubcoreMesh(core_axis_name="c", subcore_axis_name="s")
    return pl.kernel(
        _body,
        out_type=jax.ShapeDtypeStruct((_B, _D), jnp.float32),
        mesh=mesh,
        compiler_params=pltpu.CompilerParams(needs_layout_passes=False,
                                             use_tc_tiling_on_sc=False),
        scratch_types=[
            pltpu.VMEM((_BPW * _K,), jnp.int32),
            pltpu.VMEM((_K, _CHUNK), jnp.int32),
            pltpu.VMEM((_K, _CHUNK, _D), jnp.float32),
            pltpu.VMEM((_BPW, _D), jnp.float32),
            pltpu.VMEM((_L,), jnp.float32),
            pltpu.SemaphoreType.DMA,
        ],
    )(data_flat, table, attn16)


def kernel(data, embedding_table, attn_score):
    data_flat = data.reshape(_B * _K)
    attn16 = jnp.pad(attn_score.reshape(_K), (0, _L - _K))
    result = _emb(data_flat, embedding_table, attn16)
    return (result, attn_score)
